# baseline (speedup 1.0000x reference)
import jax
import jax.numpy as jnp
from jax.experimental import pallas as pl
from jax.experimental.pallas import tpu as pltpu

_KEEP_RATIO = 0.1
_B = 64
_H = 512
_W = 512
_N = _H * _W
_K = max(1, int(_N * _KEEP_RATIO))
_S = 4
_SHIFT = 12
_ITERS = 20


def _bce_with_logits(logits, targets):
    return (jnp.maximum(logits, 0.0) - logits * targets
            + jnp.log1p(jnp.exp(-jnp.abs(logits))))


def _topk_kernel(logits_ref, targets_ref, out_ref, loss_ref):
    x = logits_ref[:, 0, :, :]
    t = targets_ref[:, 0, :, :]
    loss_ref[...] = jnp.maximum(_bce_with_logits(x, t), 0.0)

    loss = loss_ref[...]
    kcount = jnp.int32(_K)
    lo0 = jnp.full((_S, 1, 1), -1, jnp.int32)
    hi0 = jnp.full((_S, 1, 1), 0x7F800000 >> _SHIFT, jnp.int32)

    def _upper_edge(code):
        return jax.lax.bitcast_convert_type((code + 1) << _SHIFT, jnp.float32)

    ones_col = jnp.ones((_W, 1), jnp.float32)

    def _count_ge(thr):
        mask = (loss >= thr).astype(jnp.float32)
        rows = jax.lax.dot_general(
            mask.reshape(_S * _H, _W), ones_col,
            (((1,), (0,)), ((), ())),
            preferred_element_type=jnp.float32)
        return jnp.sum(rows.reshape(_S, _H, 1), axis=(1, 2), keepdims=True)

    def body(_, carry):
        lo, hi = carry
        mid = lo + jax.lax.shift_right_logical(hi - lo, 1)
        cnt = _count_ge(_upper_edge(mid))
        keep_lo = cnt >= jnp.float32(_K)
        lo = jnp.where(keep_lo, mid, lo)
        hi = jnp.where(keep_lo, hi, mid)
        return lo, hi

    _, tcode = jax.lax.fori_loop(0, _ITERS, body, (lo0, hi0))

    ub = _upper_edge(tcode)
    ge = loss >= ub
    cnt_ge = jnp.sum(ge.astype(jnp.float32), axis=(1, 2), keepdims=True)
    sum_ge = jnp.sum(jnp.where(ge, loss, 0.0), axis=(1, 2), keepdims=True)
    tval = jax.lax.bitcast_convert_type(tcode << _SHIFT, jnp.float32)
    part = jnp.sum(sum_ge + tval * (jnp.float32(_K) - cnt_ge))
    out_ref[...] = jnp.reshape(part, (1, 1, 1))


def kernel(logits, targets):
    out = pl.pallas_call(
        _topk_kernel,
        grid=(_B // _S,),
        in_specs=[
            pl.BlockSpec((_S, 1, _H, _W), lambda b: (b, 0, 0, 0)),
            pl.BlockSpec((_S, 1, _H, _W), lambda b: (b, 0, 0, 0)),
        ],
        out_specs=pl.BlockSpec((1, 1, 1), lambda b: (b, 0, 0)),
        out_shape=jax.ShapeDtypeStruct((_B // _S, 1, 1), jnp.float32),
        scratch_shapes=[pltpu.VMEM((_S, _H, _W), jnp.float32)],
        compiler_params=pltpu.CompilerParams(
            dimension_semantics=("arbitrary",)),
    )(logits, targets)
    return jnp.sum(out) / jnp.float32(_B * _K)


# speedup vs baseline: 1.1678x; 1.1678x over previous
import jax
import jax.numpy as jnp
from jax.experimental import pallas as pl
from jax.experimental.pallas import tpu as pltpu

_KEEP_RATIO = 0.1
_B = 64
_H = 512
_W = 512
_N = _H * _W
_K = max(1, int(_N * _KEEP_RATIO))
_S = 4
_SHIFT = 12
_ITERS = 20


def _bce_with_logits(logits, targets):
    return (jnp.maximum(logits, 0.0) - logits * targets
            + jnp.log1p(jnp.exp(-jnp.abs(logits))))


def _topk_kernel(logits_ref, targets_ref, out_ref, loss_ref):
    x = logits_ref[:, 0, :, :]
    t = targets_ref[:, 0, :, :]
    loss_ref[...] = jnp.maximum(_bce_with_logits(x, t), 0.0)

    loss = loss_ref[...]
    kcount = jnp.int32(_K)
    lo0 = jnp.full((_S, 1, 1), -1, jnp.int32)
    hi0 = jnp.full((_S, 1, 1), 0x7F800000 >> _SHIFT, jnp.int32)

    def _upper_edge(code):
        return jax.lax.bitcast_convert_type((code + 1) << _SHIFT, jnp.float32)

    def _count_ge(thr):
        return jnp.sum(jnp.where(loss >= thr, 1.0, 0.0), axis=(1, 2),
                       keepdims=True)

    def body(_, carry):
        lo, hi = carry
        mid = lo + jax.lax.shift_right_logical(hi - lo, 1)
        cnt = _count_ge(_upper_edge(mid))
        keep_lo = cnt >= jnp.float32(_K)
        lo = jnp.where(keep_lo, mid, lo)
        hi = jnp.where(keep_lo, hi, mid)
        return lo, hi

    _, tcode = jax.lax.fori_loop(0, _ITERS, body, (lo0, hi0))

    ub = _upper_edge(tcode)
    ge = loss >= ub
    cnt_ge = jnp.sum(ge.astype(jnp.float32), axis=(1, 2), keepdims=True)
    sum_ge = jnp.sum(jnp.where(ge, loss, 0.0), axis=(1, 2), keepdims=True)
    tval = jax.lax.bitcast_convert_type(tcode << _SHIFT, jnp.float32)
    part = jnp.sum(sum_ge + tval * (jnp.float32(_K) - cnt_ge))
    out_ref[...] = jnp.reshape(part, (1, 1, 1))


def kernel(logits, targets):
    out = pl.pallas_call(
        _topk_kernel,
        grid=(_B // _S,),
        in_specs=[
            pl.BlockSpec((_S, 1, _H, _W), lambda b: (b, 0, 0, 0)),
            pl.BlockSpec((_S, 1, _H, _W), lambda b: (b, 0, 0, 0)),
        ],
        out_specs=pl.BlockSpec((1, 1, 1), lambda b: (b, 0, 0)),
        out_shape=jax.ShapeDtypeStruct((_B // _S, 1, 1), jnp.float32),
        scratch_shapes=[pltpu.VMEM((_S, _H, _W), jnp.float32)],
        compiler_params=pltpu.CompilerParams(
            dimension_semantics=("arbitrary",)),
    )(logits, targets)
    return jnp.sum(out) / jnp.float32(_B * _K)


# speedup vs baseline: 1.5908x; 1.3622x over previous
import jax
import jax.numpy as jnp
from jax.experimental import pallas as pl
from jax.experimental.pallas import tpu as pltpu

_KEEP_RATIO = 0.1
_B = 64
_H = 512
_W = 512
_N = _H * _W
_K = max(1, int(_N * _KEEP_RATIO))
_S = 4
_SHIFT = 14


def _bce_with_logits(logits, targets):
    return (jnp.maximum(logits, 0.0) - logits * targets
            + jnp.log1p(jnp.exp(-jnp.abs(logits))))


def _topk_kernel(logits_ref, targets_ref, out_ref, loss_ref):
    loss_ref[...] = jnp.maximum(
        _bce_with_logits(logits_ref[...], targets_ref[...]), 0.0)

    loss = loss_ref[...]
    axes = (1, 2, 3)

    def _upper_edge(code):
        return jax.lax.bitcast_convert_type((code + 1) << _SHIFT, jnp.float32)

    def _count_ge(thr):
        return jnp.sum(jnp.where(loss >= thr, 1.0, 0.0), axis=axes,
                       keepdims=True)

    lo_f = jnp.min(loss, axis=axes, keepdims=True)
    hi_f = jnp.max(loss, axis=axes, keepdims=True)
    lo0 = (jax.lax.bitcast_convert_type(lo_f, jnp.int32) >> _SHIFT) - 1
    hi0 = jax.lax.bitcast_convert_type(hi_f, jnp.int32) >> _SHIFT

    def cond(carry):
        lo, hi = carry
        return jnp.max(hi - lo) > 1

    def body(carry):
        lo, hi = carry
        mid = lo + jax.lax.shift_right_logical(hi - lo, 1)
        cnt = _count_ge(_upper_edge(mid))
        keep_lo = cnt >= jnp.float32(_K)
        lo = jnp.where(keep_lo, mid, lo)
        hi = jnp.where(keep_lo, hi, mid)
        return lo, hi

    _, tcode = jax.lax.while_loop(cond, body, (lo0, hi0))

    ub = _upper_edge(tcode)
    ge = loss >= ub
    cnt_ge = jnp.sum(ge.astype(jnp.float32), axis=axes, keepdims=True)
    sum_ge = jnp.sum(jnp.where(ge, loss, 0.0), axis=axes, keepdims=True)
    tval = jax.lax.bitcast_convert_type(tcode << _SHIFT, jnp.float32)
    part = jnp.sum(sum_ge + tval * (jnp.float32(_K) - cnt_ge))
    out_ref[...] = jnp.reshape(part, (1, 1, 1))


def kernel(logits, targets):
    out = pl.pallas_call(
        _topk_kernel,
        grid=(_B // _S,),
        in_specs=[
            pl.BlockSpec((_S, 1, _H, _W), lambda b: (b, 0, 0, 0)),
            pl.BlockSpec((_S, 1, _H, _W), lambda b: (b, 0, 0, 0)),
        ],
        out_specs=pl.BlockSpec((1, 1, 1), lambda b: (b, 0, 0)),
        out_shape=jax.ShapeDtypeStruct((_B // _S, 1, 1), jnp.float32),
        scratch_shapes=[pltpu.VMEM((_S, 1, _H, _W), jnp.float32)],
        compiler_params=pltpu.CompilerParams(
            dimension_semantics=("arbitrary",)),
    )(logits, targets)
    return jnp.sum(out) / jnp.float32(_B * _K)


# speedup vs baseline: 2.0095x; 1.2632x over previous
import jax
import jax.numpy as jnp
from jax.experimental import pallas as pl
from jax.experimental.pallas import tpu as pltpu

_KEEP_RATIO = 0.1
_B = 64
_H = 512
_W = 512
_N = _H * _W
_K = max(1, int(_N * _KEEP_RATIO))
_S = 4


def _bce_with_logits(logits, targets):
    return (jnp.maximum(logits, 0.0) - logits * targets
            + jnp.log1p(jnp.exp(-jnp.abs(logits))))


def _topk_kernel(logits_ref, targets_ref, out_ref, loss_ref):
    loss_ref[...] = jnp.maximum(
        _bce_with_logits(logits_ref[...], targets_ref[...]),
        0.0).astype(jnp.bfloat16)

    loss = loss_ref[...]
    axes = (1, 2, 3)
    bf_one = jnp.bfloat16(1.0)
    bf_zero = jnp.bfloat16(0.0)

    def _edge(code):
        f32 = jax.lax.bitcast_convert_type(code << 16, jnp.float32)
        return f32.astype(jnp.bfloat16)

    def _count_ge(thr):
        mask = jnp.where(loss >= thr, bf_one, bf_zero)
        part = jnp.sum(mask.reshape(_S, 1, 2, _H // 2, _W), axis=3,
                       dtype=jnp.bfloat16)
        return jnp.sum(part.astype(jnp.float32), axis=(1, 2, 3),
                       keepdims=False).reshape(_S, 1, 1, 1)

    lo_f = jnp.min(loss, axis=axes, keepdims=True).astype(jnp.float32)
    hi_f = jnp.max(loss, axis=axes, keepdims=True).astype(jnp.float32)
    lo0 = (jax.lax.bitcast_convert_type(lo_f, jnp.int32) >> 16) - 1
    hi0 = jax.lax.bitcast_convert_type(hi_f, jnp.int32) >> 16

    def cond(carry):
        lo, hi = carry
        return jnp.max(hi - lo) > 1

    def body(carry):
        lo, hi = carry
        mid = lo + jax.lax.shift_right_logical(hi - lo, 1)
        cnt = _count_ge(_edge(mid + 1))
        keep_lo = cnt >= jnp.float32(_K)
        lo = jnp.where(keep_lo, mid, lo)
        hi = jnp.where(keep_lo, hi, mid)
        return lo, hi

    _, tcode = jax.lax.while_loop(cond, body, (lo0, hi0))

    ub = _edge(tcode + 1)
    ge = loss >= ub
    cnt_ge = _count_ge(ub).reshape(_S, 1, 1, 1)
    sum_ge = jnp.sum(jnp.where(ge, loss, bf_zero).astype(jnp.float32),
                     axis=axes, keepdims=True)
    tval = jax.lax.bitcast_convert_type(tcode << 16, jnp.float32)
    part = jnp.sum(sum_ge + tval * (jnp.float32(_K) - cnt_ge))
    out_ref[...] = jnp.reshape(part, (1, 1, 1))


def kernel(logits, targets):
    out = pl.pallas_call(
        _topk_kernel,
        grid=(_B // _S,),
        in_specs=[
            pl.BlockSpec((_S, 1, _H, _W), lambda b: (b, 0, 0, 0)),
            pl.BlockSpec((_S, 1, _H, _W), lambda b: (b, 0, 0, 0)),
        ],
        out_specs=pl.BlockSpec((1, 1, 1), lambda b: (b, 0, 0)),
        out_shape=jax.ShapeDtypeStruct((_B // _S, 1, 1), jnp.float32),
        scratch_shapes=[pltpu.VMEM((_S, 1, _H, _W), jnp.bfloat16)],
        compiler_params=pltpu.CompilerParams(
            dimension_semantics=("arbitrary",)),
    )(logits, targets)
    return jnp.sum(out) / jnp.float32(_B * _K)
